# baseline (speedup 1.0000x reference)
import functools

import jax
import jax.numpy as jnp
from jax import lax
from jax.experimental import pallas as pl
from jax.experimental.pallas import tpu as pltpu
from jax.experimental.pallas import tpu_sc as plsc

D_MODEL = 4096
NUM_EXPERTS = 64
TOP_K = 8
BLOCK = 512
NBUF = 4
TOKENS = 32768
NWORK = 32
TOKW = TOKENS // NWORK
LANES = 16


def _gate_kernel(x_hbm, w_ref, b_ref, st_ref, xbuf, sems, nblocks):
    i = pl.program_id(0)

    def copy(blk):
        slot = jax.lax.rem(blk, NBUF)
        return pltpu.make_async_copy(
            x_hbm.at[pl.ds(blk * BLOCK, BLOCK), :],
            xbuf.at[slot],
            sems.at[slot],
        )

    @pl.when(i == 0)
    def _():
        for b0 in range(NBUF):
            copy(b0).start()

    @pl.when((i > 0) & (i + NBUF - 1 < nblocks))
    def _():
        copy(i + NBUF - 1).start()

    copy(i).wait()

    slot = jax.lax.rem(i, NBUF)
    x = xbuf[slot].astype(jnp.bfloat16)
    w = w_ref[...].astype(jnp.bfloat16)
    logits = jax.lax.dot_general(
        x, w, (((1,), (1,)), ((), ())),
        preferred_element_type=jnp.float32,
    ) + b_ref[...]
    m = jnp.max(logits, axis=-1, keepdims=True)
    e = jnp.exp(logits - m)
    p = e / jnp.sum(e, axis=-1, keepdims=True)
    st_ref[...] = p.T


def _scores_t(x, W, b):
    tokens = x.shape[0]
    nblocks = tokens // BLOCK
    b2 = b.reshape(1, NUM_EXPERTS)
    return pl.pallas_call(
        functools.partial(_gate_kernel, nblocks=nblocks),
        grid=(nblocks,),
        in_specs=[
            pl.BlockSpec(memory_space=pl.ANY),
            pl.BlockSpec((NUM_EXPERTS, D_MODEL), lambda i: (0, 0)),
            pl.BlockSpec((1, NUM_EXPERTS), lambda i: (0, 0)),
        ],
        out_specs=pl.BlockSpec((NUM_EXPERTS, BLOCK), lambda i: (0, i)),
        out_shape=jax.ShapeDtypeStruct((NUM_EXPERTS, tokens), jnp.float32),
        scratch_shapes=[
            pltpu.VMEM((NBUF, BLOCK, D_MODEL), jnp.float32),
            pltpu.SemaphoreType.DMA((NBUF,)),
        ],
        compiler_params=pltpu.CompilerParams(
            dimension_semantics=("arbitrary",),
        ),
    )(x, W, b2)


def _sc_topk_kernel(st_hbm, vals_hbm, idx_hbm, sbuf, vbuf, ibuf):
    wid = lax.axis_index("s") * 2 + lax.axis_index("c")
    base = wid * TOKW
    pltpu.sync_copy(st_hbm.at[:, pl.ds(base, TOKW)], sbuf)

    def group_body(g, _):
        g16 = g * LANES

        def expert_body(e, carry):
            vs = list(carry[0])
            ix = list(carry[1])
            cur_v = sbuf[e, pl.ds(g16, LANES)]
            cur_i = jnp.full((LANES,), 0, jnp.int32) + e
            for k in range(TOP_K):
                c = cur_v > vs[k]
                nv = jnp.where(c, cur_v, vs[k])
                ni = jnp.where(c, cur_i, ix[k])
                cur_v = jnp.where(c, vs[k], cur_v)
                cur_i = jnp.where(c, ix[k], cur_i)
                vs[k] = nv
                ix[k] = ni
            return (tuple(vs), tuple(ix))

        init = (
            tuple(jnp.full((LANES,), -1.0, jnp.float32) for _ in range(TOP_K)),
            tuple(jnp.full((LANES,), 0, jnp.int32) for _ in range(TOP_K)),
        )
        vs, ix = lax.fori_loop(0, NUM_EXPERTS, expert_body, init)
        for k in range(TOP_K):
            vbuf[k, pl.ds(g16, LANES)] = vs[k]
            ibuf[k, pl.ds(g16, LANES)] = ix[k]
        return 0

    lax.fori_loop(0, TOKW // LANES, group_body, 0)
    pltpu.sync_copy(vbuf, vals_hbm.at[:, pl.ds(base, TOKW)])
    pltpu.sync_copy(ibuf, idx_hbm.at[:, pl.ds(base, TOKW)])


@jax.jit
def kernel(x, W, b):
    st = _scores_t(x, W, b)
    mesh = plsc.VectorSubcoreMesh(core_axis_name="c", subcore_axis_name="s")
    sc = functools.partial(
        pl.kernel,
        mesh=mesh,
        out_type=[
            jax.ShapeDtypeStruct((TOP_K, TOKENS), jnp.float32),
            jax.ShapeDtypeStruct((TOP_K, TOKENS), jnp.int32),
        ],
        scratch_types=[
            pltpu.VMEM((NUM_EXPERTS, TOKW), jnp.float32),
            pltpu.VMEM((TOP_K, TOKW), jnp.float32),
            pltpu.VMEM((TOP_K, TOKW), jnp.int32),
        ],
    )(_sc_topk_kernel)
    vals_t, idx_t = sc(st)
    return vals_t.T, idx_t.T


# speedup vs baseline: 1.1444x; 1.1444x over previous
import functools

import jax
import jax.numpy as jnp
from jax.experimental import pallas as pl
from jax.experimental.pallas import tpu as pltpu

D_MODEL = 4096
NUM_EXPERTS = 64
TOP_K = 8
BLOCK = 256
NBUF = 8


def _gate_kernel(x_hbm, w_ref, b_ref, vals_ref, idx_ref, xbuf, sems, nblocks):
    i = pl.program_id(0)

    def copy(blk):
        slot = jax.lax.rem(blk, NBUF)
        return pltpu.make_async_copy(
            x_hbm.at[pl.ds(blk * BLOCK, BLOCK), :],
            xbuf.at[slot],
            sems.at[slot],
        )

    @pl.when(i == 0)
    def _():
        for b0 in range(NBUF):
            copy(b0).start()

    @pl.when((i > 0) & (i + NBUF - 1 < nblocks))
    def _():
        copy(i + NBUF - 1).start()

    copy(i).wait()

    slot = jax.lax.rem(i, NBUF)
    x = xbuf[slot].astype(jnp.bfloat16)
    w = w_ref[...].astype(jnp.bfloat16)
    logits = jax.lax.dot_general(
        x, w, (((1,), (1,)), ((), ())),
        preferred_element_type=jnp.float32,
    ) + b_ref[...]
    m = jnp.max(logits, axis=-1, keepdims=True)
    e = jnp.exp(logits - m)
    p = e / jnp.sum(e, axis=-1, keepdims=True)

    s = p.T
    iota = jax.lax.broadcasted_iota(jnp.int32, s.shape, 0)
    vals = []
    idxs = []
    for _ in range(TOP_K):
        mk = jnp.max(s, axis=0, keepdims=True)
        ik = jnp.min(jnp.where(s == mk, iota, NUM_EXPERTS), axis=0, keepdims=True)
        vals.append(mk)
        idxs.append(ik)
        s = jnp.where(iota == ik, -1.0, s)
    vals_ref[...] = jnp.concatenate(vals, axis=0).T
    idx_ref[...] = jnp.concatenate(idxs, axis=0).T


@jax.jit
def kernel(x, W, b):
    tokens = x.shape[0]
    nblocks = tokens // BLOCK
    b2 = b.reshape(1, NUM_EXPERTS)
    vals, idx = pl.pallas_call(
        functools.partial(_gate_kernel, nblocks=nblocks),
        grid=(nblocks,),
        in_specs=[
            pl.BlockSpec(memory_space=pl.ANY),
            pl.BlockSpec((NUM_EXPERTS, D_MODEL), lambda i: (0, 0)),
            pl.BlockSpec((1, NUM_EXPERTS), lambda i: (0, 0)),
        ],
        out_specs=[
            pl.BlockSpec((BLOCK, TOP_K), lambda i: (i, 0)),
            pl.BlockSpec((BLOCK, TOP_K), lambda i: (i, 0)),
        ],
        out_shape=[
            jax.ShapeDtypeStruct((tokens, TOP_K), jnp.float32),
            jax.ShapeDtypeStruct((tokens, TOP_K), jnp.int32),
        ],
        scratch_shapes=[
            pltpu.VMEM((NBUF, BLOCK, D_MODEL), jnp.float32),
            pltpu.SemaphoreType.DMA((NBUF,)),
        ],
        compiler_params=pltpu.CompilerParams(
            dimension_semantics=("arbitrary",),
        ),
    )(x, W, b2)
    return vals, idx


# speedup vs baseline: 1.1484x; 1.0035x over previous
import functools

import jax
import jax.numpy as jnp
from jax.experimental import pallas as pl
from jax.experimental.pallas import tpu as pltpu

D_MODEL = 4096
NUM_EXPERTS = 64
TOP_K = 8
BLOCK = 1024
NBUF = 3


def _gate_kernel(x_hbm, w_ref, b_ref, vals_ref, idx_ref, xbuf, sems, nblocks):
    i = pl.program_id(0)

    def copy(blk):
        slot = jax.lax.rem(blk, NBUF)
        return pltpu.make_async_copy(
            x_hbm.at[pl.ds(blk * BLOCK, BLOCK), :],
            xbuf.at[slot],
            sems.at[slot],
        )

    @pl.when(i == 0)
    def _():
        for b0 in range(NBUF):
            copy(b0).start()

    @pl.when((i > 0) & (i + NBUF - 1 < nblocks))
    def _():
        copy(i + NBUF - 1).start()

    copy(i).wait()

    slot = jax.lax.rem(i, NBUF)
    x = xbuf[slot].astype(jnp.bfloat16)
    w = w_ref[...].astype(jnp.bfloat16)
    logits = jax.lax.dot_general(
        x, w, (((1,), (1,)), ((), ())),
        preferred_element_type=jnp.float32,
    ) + b_ref[...]
    m = jnp.max(logits, axis=-1, keepdims=True)
    e = jnp.exp(logits - m)
    p = e / jnp.sum(e, axis=-1, keepdims=True)

    s = p.T
    iota = jax.lax.broadcasted_iota(jnp.int32, s.shape, 0)
    vals = []
    idxs = []
    for _ in range(TOP_K):
        mk = jnp.max(s, axis=0, keepdims=True)
        ik = jnp.min(jnp.where(s == mk, iota, NUM_EXPERTS), axis=0, keepdims=True)
        vals.append(mk)
        idxs.append(ik)
        s = jnp.where(iota == ik, -1.0, s)
    vals_ref[...] = jnp.concatenate(vals, axis=0).T
    idx_ref[...] = jnp.concatenate(idxs, axis=0).T


@jax.jit
def kernel(x, W, b):
    tokens = x.shape[0]
    nblocks = tokens // BLOCK
    b2 = b.reshape(1, NUM_EXPERTS)
    vals, idx = pl.pallas_call(
        functools.partial(_gate_kernel, nblocks=nblocks),
        grid=(nblocks,),
        in_specs=[
            pl.BlockSpec(memory_space=pl.ANY),
            pl.BlockSpec((NUM_EXPERTS, D_MODEL), lambda i: (0, 0)),
            pl.BlockSpec((1, NUM_EXPERTS), lambda i: (0, 0)),
        ],
        out_specs=[
            pl.BlockSpec((BLOCK, TOP_K), lambda i: (i, 0)),
            pl.BlockSpec((BLOCK, TOP_K), lambda i: (i, 0)),
        ],
        out_shape=[
            jax.ShapeDtypeStruct((tokens, TOP_K), jnp.float32),
            jax.ShapeDtypeStruct((tokens, TOP_K), jnp.int32),
        ],
        scratch_shapes=[
            pltpu.VMEM((NBUF, BLOCK, D_MODEL), jnp.float32),
            pltpu.SemaphoreType.DMA((NBUF,)),
        ],
        compiler_params=pltpu.CompilerParams(
            dimension_semantics=("arbitrary",),
        ),
    )(x, W, b2)
    return vals, idx


# speedup vs baseline: 1.1538x; 1.0047x over previous
import functools

import jax
import jax.numpy as jnp
from jax.experimental import pallas as pl
from jax.experimental.pallas import tpu as pltpu

D_MODEL = 4096
NUM_EXPERTS = 64
TOP_K = 8
BLOCK = 512
NBUF = 6


def _gate_kernel(x_hbm, w_ref, b_ref, vals_ref, idx_ref, xbuf, sems, nblocks):
    i = pl.program_id(0)

    def copy(blk):
        slot = jax.lax.rem(blk, NBUF)
        return pltpu.make_async_copy(
            x_hbm.at[pl.ds(blk * BLOCK, BLOCK), :],
            xbuf.at[slot],
            sems.at[slot],
        )

    @pl.when(i == 0)
    def _():
        for b0 in range(NBUF):
            copy(b0).start()

    @pl.when((i > 0) & (i + NBUF - 1 < nblocks))
    def _():
        copy(i + NBUF - 1).start()

    copy(i).wait()

    slot = jax.lax.rem(i, NBUF)
    x = xbuf[slot].astype(jnp.bfloat16)
    w = w_ref[...].astype(jnp.bfloat16)
    logits = jax.lax.dot_general(
        x, w, (((1,), (1,)), ((), ())),
        preferred_element_type=jnp.float32,
    ) + b_ref[...]
    m = jnp.max(logits, axis=-1, keepdims=True)
    e = jnp.exp(logits - m)
    p = e / jnp.sum(e, axis=-1, keepdims=True)

    s = p.T
    iota = jax.lax.broadcasted_iota(jnp.int32, s.shape, 0)
    vals = []
    idxs = []
    for _ in range(TOP_K):
        mk = jnp.max(s, axis=0, keepdims=True)
        ik = jnp.min(jnp.where(s == mk, iota, NUM_EXPERTS), axis=0, keepdims=True)
        vals.append(mk)
        idxs.append(ik)
        s = jnp.where(iota == ik, -1.0, s)
    vals_ref[...] = jnp.concatenate(vals, axis=0).T
    idx_ref[...] = jnp.concatenate(idxs, axis=0).T


@jax.jit
def kernel(x, W, b):
    tokens = x.shape[0]
    nblocks = tokens // BLOCK
    b2 = b.reshape(1, NUM_EXPERTS)
    vals, idx = pl.pallas_call(
        functools.partial(_gate_kernel, nblocks=nblocks),
        grid=(nblocks,),
        in_specs=[
            pl.BlockSpec(memory_space=pl.ANY),
            pl.BlockSpec((NUM_EXPERTS, D_MODEL), lambda i: (0, 0)),
            pl.BlockSpec((1, NUM_EXPERTS), lambda i: (0, 0)),
        ],
        out_specs=[
            pl.BlockSpec((BLOCK, TOP_K), lambda i: (i, 0)),
            pl.BlockSpec((BLOCK, TOP_K), lambda i: (i, 0)),
        ],
        out_shape=[
            jax.ShapeDtypeStruct((tokens, TOP_K), jnp.float32),
            jax.ShapeDtypeStruct((tokens, TOP_K), jnp.int32),
        ],
        scratch_shapes=[
            pltpu.VMEM((NBUF, BLOCK, D_MODEL), jnp.float32),
            pltpu.SemaphoreType.DMA((NBUF,)),
        ],
        compiler_params=pltpu.CompilerParams(
            dimension_semantics=("arbitrary",),
        ),
    )(x, W, b2)
    return vals, idx
